# Initial kernel scaffold; baseline (speedup 1.0000x reference)
import jax
import jax.numpy as jnp
from jax.experimental import pallas as pl


def kernel(proposal_embeddings, proposal_objectness, class_embeddings, proposal_bboxes, image_ids):
    raise NotImplementedError("write your pallas kernel here")


# speedup vs baseline: 11.5110x; 11.5110x over previous
import functools

import jax
import jax.numpy as jnp
from jax.experimental import pallas as pl
from jax.experimental.pallas import tpu as pltpu

SOFTMAX_T = 0.01
TOPK_CLIP = 3
NMS_SCORE_THR = 0.05
NMS_IOU_THR = 0.5
CLIP_RATIO = 0.5
OBJ_RATIO = 0.5
PRE_NMS = 1000
CLS_OFFSET = 4096.0

_B, _N, _D, _C = 4, 5000, 512, 80
_TN = 1000
_P = 1024


def _score_kernel(pe_ref, po_ref, nrm_ref, ce_ref, out_ref):
    x = pe_ref[0]
    nrmc = jnp.transpose(nrm_ref[0])
    xn = x / nrmc
    logit = jnp.dot(xn, ce_ref[...],
                    preferred_element_type=jnp.float32,
                    precision=jax.lax.Precision.DEFAULT)
    s = logit / SOFTMAX_T
    m = jnp.max(s, axis=1, keepdims=True)
    e = jnp.exp(s - m)
    v = jnp.concatenate([e, jnp.zeros((e.shape[0], 128 - e.shape[1]),
                                      jnp.float32)], axis=1)
    while v.shape[1] > 1:
        h = v.shape[1] // 2
        v = v[:, :h] + v[:, h:]
    p = e / v

    cidx = jax.lax.broadcasted_iota(jnp.int32, p.shape, 1)
    m1 = jnp.max(p, axis=1, keepdims=True)
    c1 = jnp.min(jnp.where(p == m1, cidx, _C), axis=1, keepdims=True)
    p2 = jnp.where(cidx == c1, -jnp.inf, p)
    m2 = jnp.max(p2, axis=1, keepdims=True)
    c2 = jnp.min(jnp.where(p2 == m2, cidx, _C), axis=1, keepdims=True)
    p3 = jnp.where(cidx == c2, -jnp.inf, p2)
    m3 = jnp.max(p3, axis=1, keepdims=True)

    clip_k = p * (p >= m3).astype(jnp.float32)
    obj = po_ref[0]
    objc = jnp.transpose(obj)
    final = clip_k * CLIP_RATIO + ((clip_k > 0).astype(jnp.float32) * objc) * OBJ_RATIO
    out_ref[0] = final


def _nms_kernel(rows_ref, cols_ref, keep_ref, iou_scr):
    R = rows_ref[0]
    Cc = cols_ref[0]
    offj = R[5:6, :] * CLS_OFFSET
    offi = Cc[:, 5:6] * CLS_OFFSET
    x1j = R[0:1, :] + offj
    y1j = R[1:2, :] + offj
    x2j = R[2:3, :] + offj
    y2j = R[3:4, :] + offj
    x1i = Cc[:, 0:1] + offi
    y1i = Cc[:, 1:2] + offi
    x2i = Cc[:, 2:3] + offi
    y2i = Cc[:, 3:4] + offi

    xx1 = jnp.maximum(x1i, x1j)
    yy1 = jnp.maximum(y1i, y1j)
    xx2 = jnp.minimum(x2i, x2j)
    yy2 = jnp.minimum(y2i, y2j)
    inter = jnp.maximum(xx2 - xx1, 0.0) * jnp.maximum(yy2 - yy1, 0.0)
    ai = jnp.maximum(x2i - x1i, 0.0) * jnp.maximum(y2i - y1i, 0.0)
    aj = jnp.maximum(x2j - x1j, 0.0) * jnp.maximum(y2j - y1j, 0.0)
    iou_scr[...] = inter / (ai + aj - inter + 1e-6)

    scores = R[4:5, :]
    lane = jax.lax.broadcasted_iota(jnp.int32, (1, _P), 1)
    keep0 = (scores > NMS_SCORE_THR).astype(jnp.float32)

    def body(i, keep):
        row = iou_scr[pl.ds(i, 1), :]
        kvi = jnp.sum(jnp.where(lane == i, keep, 0.0))
        sup = (row > NMS_IOU_THR) & (kvi > 0.0) & (lane > i)
        return jnp.where(sup, 0.0, keep)

    keep = jax.lax.fori_loop(0, PRE_NMS, body, keep0)
    keep_ref[0] = keep


@functools.partial(jax.jit, static_argnums=())
def kernel(proposal_embeddings, proposal_objectness, class_embeddings,
           proposal_bboxes, image_ids):
    B, N, D = proposal_embeddings.shape
    C = class_embeddings.shape[0]
    nt = N // _TN

    ce_t = jnp.transpose(class_embeddings)
    po3 = proposal_objectness.reshape(B * nt, 1, _TN)
    nrm3 = jnp.linalg.norm(proposal_embeddings, axis=2).reshape(B * nt, 1, _TN)

    final = pl.pallas_call(
        _score_kernel,
        grid=(B, nt),
        in_specs=[
            pl.BlockSpec((1, _TN, D), lambda b, n: (b, n, 0)),
            pl.BlockSpec((1, 1, _TN), lambda b, n: (b * nt + n, 0, 0)),
            pl.BlockSpec((1, 1, _TN), lambda b, n: (b * nt + n, 0, 0)),
            pl.BlockSpec((D, C), lambda b, n: (0, 0)),
        ],
        out_specs=pl.BlockSpec((1, _TN, C), lambda b, n: (b, n, 0)),
        out_shape=jax.ShapeDtypeStruct((B, N, C), jnp.float32),
    )(proposal_embeddings, po3, nrm3, ce_t)

    flat = final.reshape(B, N * C)
    scores, idx = jax.lax.top_k(flat, PRE_NMS)
    box_idx = idx // C
    cls = idx % C
    b = jnp.take_along_axis(proposal_bboxes, box_idx[..., None], axis=1)

    fields = jnp.concatenate(
        [b, scores[..., None], cls.astype(jnp.float32)[..., None],
         jnp.zeros((B, PRE_NMS, 2), jnp.float32)], axis=2)
    cols = jnp.pad(fields, ((0, 0), (0, _P - PRE_NMS), (0, 0)))
    rows = jnp.transpose(cols, (0, 2, 1))

    keepf = pl.pallas_call(
        _nms_kernel,
        grid=(B,),
        in_specs=[
            pl.BlockSpec((1, 8, _P), lambda i: (i, 0, 0)),
            pl.BlockSpec((1, _P, 8), lambda i: (i, 0, 0)),
        ],
        out_specs=pl.BlockSpec((1, 1, _P), lambda i: (i, 0, 0)),
        out_shape=jax.ShapeDtypeStruct((B, 1, _P), jnp.float32),
        scratch_shapes=[pltpu.VMEM((_P, _P), jnp.float32)],
    )(rows, cols)

    keep = keepf[:, 0, :PRE_NMS] > 0.5
    dets = jnp.where(keep[..., None],
                     jnp.concatenate([b, scores[..., None]], axis=2), 0.0)
    labels = jnp.where(keep, cls, -1)
    imgs = jnp.where(keep, image_ids[:, None], -1)
    return (dets.reshape(B * PRE_NMS, 5),
            labels.reshape(B * PRE_NMS),
            imgs.reshape(B * PRE_NMS))


# speedup vs baseline: 14.3580x; 1.2473x over previous
import functools

import jax
import jax.numpy as jnp
from jax.experimental import pallas as pl
from jax.experimental.pallas import tpu as pltpu

SOFTMAX_T = 0.01
TOPK_CLIP = 3
NMS_SCORE_THR = 0.05
NMS_IOU_THR = 0.5
CLIP_RATIO = 0.5
OBJ_RATIO = 0.5
PRE_NMS = 1000
CLS_OFFSET = 4096.0

_B, _N, _D, _C = 4, 5000, 512, 80
_TN = 1000
_P = 1024


def _score_kernel(pe_ref, po_ref, nrm_ref, ce_ref, out_ref):
    x = pe_ref[0]
    nrmc = jnp.transpose(nrm_ref[0])
    xn = x / nrmc
    logit = jnp.dot(xn, ce_ref[...],
                    preferred_element_type=jnp.float32,
                    precision=jax.lax.Precision.DEFAULT)
    s = logit / SOFTMAX_T
    m = jnp.max(s, axis=1, keepdims=True)
    e = jnp.exp(s - m)
    v = jnp.concatenate([e, jnp.zeros((e.shape[0], 128 - e.shape[1]),
                                      jnp.float32)], axis=1)
    while v.shape[1] > 1:
        h = v.shape[1] // 2
        v = v[:, :h] + v[:, h:]
    p = e / v

    cidx = jax.lax.broadcasted_iota(jnp.int32, p.shape, 1)
    m1 = jnp.max(p, axis=1, keepdims=True)
    c1 = jnp.min(jnp.where(p == m1, cidx, _C), axis=1, keepdims=True)
    p2 = jnp.where(cidx == c1, -jnp.inf, p)
    m2 = jnp.max(p2, axis=1, keepdims=True)
    c2 = jnp.min(jnp.where(p2 == m2, cidx, _C), axis=1, keepdims=True)
    p3 = jnp.where(cidx == c2, -jnp.inf, p2)
    m3 = jnp.max(p3, axis=1, keepdims=True)

    clip_k = p * (p >= m3).astype(jnp.float32)
    obj = po_ref[0]
    objc = jnp.transpose(obj)
    final = clip_k * CLIP_RATIO + ((clip_k > 0).astype(jnp.float32) * objc) * OBJ_RATIO
    out_ref[0] = final


def _nms_kernel(rows_ref, cols_ref, keep_ref, s0, s1, s2, s3):
    scr = [s0, s1, s2, s3]
    keep0_rows = []
    for b_ in range(_B):
        R = rows_ref[b_]
        Cc = cols_ref[b_]
        offj = R[5:6, :] * CLS_OFFSET
        offi = Cc[:, 5:6] * CLS_OFFSET
        x1j = R[0:1, :] + offj
        y1j = R[1:2, :] + offj
        x2j = R[2:3, :] + offj
        y2j = R[3:4, :] + offj
        x1i = Cc[:, 0:1] + offi
        y1i = Cc[:, 1:2] + offi
        x2i = Cc[:, 2:3] + offi
        y2i = Cc[:, 3:4] + offi

        xx1 = jnp.maximum(x1i, x1j)
        yy1 = jnp.maximum(y1i, y1j)
        xx2 = jnp.minimum(x2i, x2j)
        yy2 = jnp.minimum(y2i, y2j)
        inter = jnp.maximum(xx2 - xx1, 0.0) * jnp.maximum(yy2 - yy1, 0.0)
        ai = jnp.maximum(x2i - x1i, 0.0) * jnp.maximum(y2i - y1i, 0.0)
        aj = jnp.maximum(x2j - x1j, 0.0) * jnp.maximum(y2j - y1j, 0.0)
        scr[b_][...] = inter / (ai + aj - inter + 1e-6)
        keep0_rows.append((R[4:5, :] > NMS_SCORE_THR).astype(jnp.float32))

    keep0 = jnp.concatenate(keep0_rows, axis=0)
    lane = jax.lax.broadcasted_iota(jnp.int32, (_B, _P), 1)

    def body(i, keep):
        rows = jnp.concatenate(
            [s[pl.ds(i, 1), :] for s in scr], axis=0)
        kvi = jnp.sum(jnp.where(lane == i, keep, 0.0),
                      axis=1, keepdims=True)
        sup = (rows > NMS_IOU_THR) & (kvi > 0.0) & (lane > i)
        return jnp.where(sup, 0.0, keep)

    keep = jax.lax.fori_loop(0, PRE_NMS, body, keep0)
    keep_ref[...] = keep


@functools.partial(jax.jit, static_argnums=())
def kernel(proposal_embeddings, proposal_objectness, class_embeddings,
           proposal_bboxes, image_ids):
    B, N, D = proposal_embeddings.shape
    C = class_embeddings.shape[0]
    nt = N // _TN

    ce_t = jnp.transpose(class_embeddings)
    po3 = proposal_objectness.reshape(B * nt, 1, _TN)
    nrm3 = jnp.linalg.norm(proposal_embeddings, axis=2).reshape(B * nt, 1, _TN)

    final = pl.pallas_call(
        _score_kernel,
        grid=(B, nt),
        in_specs=[
            pl.BlockSpec((1, _TN, D), lambda b, n: (b, n, 0)),
            pl.BlockSpec((1, 1, _TN), lambda b, n: (b * nt + n, 0, 0)),
            pl.BlockSpec((1, 1, _TN), lambda b, n: (b * nt + n, 0, 0)),
            pl.BlockSpec((D, C), lambda b, n: (0, 0)),
        ],
        out_specs=pl.BlockSpec((1, _TN, C), lambda b, n: (b, n, 0)),
        out_shape=jax.ShapeDtypeStruct((B, N, C), jnp.float32),
    )(proposal_embeddings, po3, nrm3, ce_t)

    flat = final.reshape(B, N * C)
    scores, idx = jax.lax.top_k(flat, PRE_NMS)
    box_idx = idx // C
    cls = idx % C
    b = jnp.take_along_axis(proposal_bboxes, box_idx[..., None], axis=1)

    fields = jnp.concatenate(
        [b, scores[..., None], cls.astype(jnp.float32)[..., None],
         jnp.zeros((B, PRE_NMS, 2), jnp.float32)], axis=2)
    cols = jnp.pad(fields, ((0, 0), (0, _P - PRE_NMS), (0, 0)))
    rows = jnp.transpose(cols, (0, 2, 1))

    keepf = pl.pallas_call(
        _nms_kernel,
        in_specs=[
            pl.BlockSpec((B, 8, _P), lambda: (0, 0, 0)),
            pl.BlockSpec((B, _P, 8), lambda: (0, 0, 0)),
        ],
        out_specs=pl.BlockSpec((B, _P), lambda: (0, 0)),
        out_shape=jax.ShapeDtypeStruct((B, _P), jnp.float32),
        scratch_shapes=[pltpu.VMEM((_P, _P), jnp.float32)] * 4,
    )(rows, cols)

    keep = keepf[:, :PRE_NMS] > 0.5
    dets = jnp.where(keep[..., None],
                     jnp.concatenate([b, scores[..., None]], axis=2), 0.0)
    labels = jnp.where(keep, cls, -1)
    imgs = jnp.where(keep, image_ids[:, None], -1)
    return (dets.reshape(B * PRE_NMS, 5),
            labels.reshape(B * PRE_NMS),
            imgs.reshape(B * PRE_NMS))


# speedup vs baseline: 59.2722x; 4.1282x over previous
import functools

import jax
import jax.numpy as jnp
from jax.experimental import pallas as pl
from jax.experimental.pallas import tpu as pltpu

SOFTMAX_T = 0.01
TOPK_CLIP = 3
NMS_SCORE_THR = 0.05
NMS_IOU_THR = 0.5
CLIP_RATIO = 0.5
OBJ_RATIO = 0.5
PRE_NMS = 1000
CLS_OFFSET = 4096.0

_B, _N, _D, _C = 4, 5000, 512, 80
_TN = 1000
_P = 1024


def _score_kernel(pe_ref, po_ref, nrm_ref, ce_ref, out_ref, oidx_ref):
    x = pe_ref[0]
    nrmc = jnp.transpose(nrm_ref[0])
    xn = x / nrmc
    logit = jnp.dot(xn, ce_ref[...],
                    preferred_element_type=jnp.float32,
                    precision=jax.lax.Precision.DEFAULT)
    s = logit / SOFTMAX_T
    m = jnp.max(s, axis=1, keepdims=True)
    e = jnp.exp(s - m)
    v = jnp.concatenate([e, jnp.zeros((e.shape[0], 128 - e.shape[1]),
                                      jnp.float32)], axis=1)
    while v.shape[1] > 1:
        h = v.shape[1] // 2
        v = v[:, :h] + v[:, h:]
    p = e / v

    cidx = jax.lax.broadcasted_iota(jnp.int32, p.shape, 1)
    m1 = jnp.max(p, axis=1, keepdims=True)
    c1 = jnp.min(jnp.where(p == m1, cidx, _C), axis=1, keepdims=True)
    p2 = jnp.where(cidx == c1, -jnp.inf, p)
    m2 = jnp.max(p2, axis=1, keepdims=True)
    c2 = jnp.min(jnp.where(p2 == m2, cidx, _C), axis=1, keepdims=True)
    p3 = jnp.where(cidx == c2, -jnp.inf, p2)
    m3 = jnp.max(p3, axis=1, keepdims=True)
    c3 = jnp.min(jnp.where(p3 == m3, cidx, _C), axis=1, keepdims=True)

    obj = po_ref[0]
    objc = jnp.transpose(obj)

    cmin = jnp.minimum(jnp.minimum(c1, c2), c3)
    cmax = jnp.maximum(jnp.maximum(c1, c2), c3)
    cmid = c1 + c2 + c3 - cmin - cmax
    def pick(c):
        return jnp.where(c == c1, m1, jnp.where(c == c2, m2, m3))
    rows_v, rows_i = [], []
    tile = pl.program_id(1)
    ridx = jax.lax.broadcasted_iota(jnp.int32, (_TN, 1), 0)
    for c in (cmin, cmid, cmax):
        v = pick(c)
        fused = v * CLIP_RATIO + ((v > 0).astype(jnp.float32) * objc) * OBJ_RATIO
        rows_v.append(jnp.transpose(fused))
        rows_i.append(jnp.transpose(ridx * _C + c))
    out_ref[0] = jnp.concatenate(rows_v, axis=0)
    oidx_ref[0] = jnp.concatenate(rows_i, axis=0) + tile * (_TN * _C)


def _nms_kernel(rows_ref, cols_ref, keep_ref, s0, s1, s2, s3):
    scr = [s0, s1, s2, s3]
    keep0_rows = []
    for b_ in range(_B):
        R = rows_ref[b_]
        Cc = cols_ref[b_]
        offj = R[5:6, :] * CLS_OFFSET
        offi = Cc[:, 5:6] * CLS_OFFSET
        x1j = R[0:1, :] + offj
        y1j = R[1:2, :] + offj
        x2j = R[2:3, :] + offj
        y2j = R[3:4, :] + offj
        x1i = Cc[:, 0:1] + offi
        y1i = Cc[:, 1:2] + offi
        x2i = Cc[:, 2:3] + offi
        y2i = Cc[:, 3:4] + offi

        xx1 = jnp.maximum(x1i, x1j)
        yy1 = jnp.maximum(y1i, y1j)
        xx2 = jnp.minimum(x2i, x2j)
        yy2 = jnp.minimum(y2i, y2j)
        inter = jnp.maximum(xx2 - xx1, 0.0) * jnp.maximum(yy2 - yy1, 0.0)
        ai = jnp.maximum(x2i - x1i, 0.0) * jnp.maximum(y2i - y1i, 0.0)
        aj = jnp.maximum(x2j - x1j, 0.0) * jnp.maximum(y2j - y1j, 0.0)
        scr[b_][...] = inter / (ai + aj - inter + 1e-6)
        keep0_rows.append((R[4:5, :] > NMS_SCORE_THR).astype(jnp.float32))

    keep0 = jnp.concatenate(keep0_rows, axis=0)
    lane = jax.lax.broadcasted_iota(jnp.int32, (_B, _P), 1)

    def body(i, keep):
        rows = jnp.concatenate(
            [s[pl.ds(i, 1), :] for s in scr], axis=0)
        kvi = jnp.sum(jnp.where(lane == i, keep, 0.0),
                      axis=1, keepdims=True)
        sup = (rows > NMS_IOU_THR) & (kvi > 0.0) & (lane > i)
        return jnp.where(sup, 0.0, keep)

    keep = jax.lax.fori_loop(0, PRE_NMS, body, keep0)
    keep_ref[...] = keep


@functools.partial(jax.jit, static_argnums=())
def kernel(proposal_embeddings, proposal_objectness, class_embeddings,
           proposal_bboxes, image_ids):
    B, N, D = proposal_embeddings.shape
    C = class_embeddings.shape[0]
    nt = N // _TN

    ce_t = jnp.transpose(class_embeddings)
    po3 = proposal_objectness.reshape(B * nt, 1, _TN)
    nrm3 = jnp.linalg.norm(proposal_embeddings, axis=2).reshape(B * nt, 1, _TN)

    final = pl.pallas_call(
        _score_kernel,
        grid=(B, nt),
        in_specs=[
            pl.BlockSpec((1, _TN, D), lambda b, n: (b, n, 0)),
            pl.BlockSpec((1, 1, _TN), lambda b, n: (b * nt + n, 0, 0)),
            pl.BlockSpec((1, 1, _TN), lambda b, n: (b * nt + n, 0, 0)),
            pl.BlockSpec((D, C), lambda b, n: (0, 0)),
        ],
        out_specs=[
            pl.BlockSpec((1, 3, _TN), lambda b, n: (b * nt + n, 0, 0)),
            pl.BlockSpec((1, 3, _TN), lambda b, n: (b * nt + n, 0, 0)),
        ],
        out_shape=[
            jax.ShapeDtypeStruct((B * nt, 3, _TN), jnp.float32),
            jax.ShapeDtypeStruct((B * nt, 3, _TN), jnp.int32),
        ],
    )(proposal_embeddings, po3, nrm3, ce_t)
    cand_v, cand_i = final

    vals = cand_v.reshape(B, nt, 3, _TN).transpose(0, 1, 3, 2).reshape(B, nt * _TN * 3)
    idxs = cand_i.reshape(B, nt, 3, _TN).transpose(0, 1, 3, 2).reshape(B, nt * _TN * 3)
    scores, pos = jax.lax.top_k(vals, PRE_NMS)
    idx = jnp.take_along_axis(idxs, pos, axis=1)
    box_idx = idx // C
    cls = idx % C
    b = jnp.take_along_axis(proposal_bboxes, box_idx[..., None], axis=1)

    fields = jnp.concatenate(
        [b, scores[..., None], cls.astype(jnp.float32)[..., None],
         jnp.zeros((B, PRE_NMS, 2), jnp.float32)], axis=2)
    cols = jnp.pad(fields, ((0, 0), (0, _P - PRE_NMS), (0, 0)))
    rows = jnp.transpose(cols, (0, 2, 1))

    keepf = pl.pallas_call(
        _nms_kernel,
        in_specs=[
            pl.BlockSpec((B, 8, _P), lambda: (0, 0, 0)),
            pl.BlockSpec((B, _P, 8), lambda: (0, 0, 0)),
        ],
        out_specs=pl.BlockSpec((B, _P), lambda: (0, 0)),
        out_shape=jax.ShapeDtypeStruct((B, _P), jnp.float32),
        scratch_shapes=[pltpu.VMEM((_P, _P), jnp.float32)] * 4,
    )(rows, cols)

    keep = keepf[:, :PRE_NMS] > 0.5
    dets = jnp.where(keep[..., None],
                     jnp.concatenate([b, scores[..., None]], axis=2), 0.0)
    labels = jnp.where(keep, cls, -1)
    imgs = jnp.where(keep, image_ids[:, None], -1)
    return (dets.reshape(B * PRE_NMS, 5),
            labels.reshape(B * PRE_NMS),
            imgs.reshape(B * PRE_NMS))
